# baseline (speedup 1.0000x reference)
import functools
import jax
import jax.numpy as jnp
from jax import lax
from jax.experimental import pallas as pl
from jax.experimental.pallas import tpu as pltpu
from jax.experimental.pallas import tpu_sc as plsc

N_USERS = 30000
N_ITEMS = 20000
N_NODES = N_USERS + N_ITEMS
EMB = 64
N_EDGE_HALF = 400000
BATCH = 4096
TAU = 0.2
REG_LAMBDA = 1e-4
SSL_LAMBDA = 0.1

NC, NS = 2, 16
CH = 128
N_EDGES = 2 * N_EDGE_HALF
FULL_CH = 390
TILE_EDGES = 49920
REM_CH = 10
HALF_NODES = 25000
ACC_ROWS = 25024
TRASH = 25000
ZR = 200

_mesh = plsc.VectorSubcoreMesh(core_axis_name="c", subcore_axis_name="s")


@functools.partial(
    pl.kernel,
    out_type=jax.ShapeDtypeStruct((N_NODES, EMB), jnp.float32),
    mesh=_mesh,
    scratch_types=[
        pltpu.VMEM((CH,), jnp.int32),
        pltpu.VMEM((CH,), jnp.int32),
        pltpu.VMEM((CH,), jnp.int32),
        pltpu.VMEM((CH,), jnp.float32),
        pltpu.VMEM((CH, EMB), jnp.float32),
        pltpu.VMEM((CH,), jnp.int32),
        pltpu.VMEM((CH,), jnp.int32),
        pltpu.VMEM((CH,), jnp.int32),
        pltpu.VMEM((CH,), jnp.float32),
        pltpu.VMEM((CH, EMB), jnp.float32),
        pltpu.VMEM((ZR, EMB), jnp.float32),
        pltpu.VMEM_SHARED((ACC_ROWS, EMB), jnp.float32),
        pltpu.SemaphoreType.DMA,
        pltpu.SemaphoreType.DMA,
        pltpu.SemaphoreType.DMA,
        pltpu.SemaphoreType.DMA,
    ],
    compiler_params=pltpu.CompilerParams(use_tc_tiling_on_sc=False),
)
def _propagate(emb_h, src_h, dst_h, vals_h, out_h,
               src0, dstr0, dst0, vals0, rows0,
               src1, dstr1, dst1, vals1, rows1,
               zbuf, acc, isem0, isem1, gsem0, gsem1):
    c = lax.axis_index("c")
    s = lax.axis_index("s")
    noff = c * HALF_NODES

    def zinit(r, _):
        for q in range(4):
            zbuf[r, pl.ds(q * 16, 16)] = jnp.zeros((16,), jnp.float32)
        return 0
    lax.fori_loop(0, ZR, zinit, 0)

    def zchunk(j, _):
        k = s + j * NS

        @pl.when(k < HALF_NODES // ZR)
        def _():
            pltpu.sync_copy(zbuf, acc.at[pl.ds(k * ZR, ZR)])
        return 0
    lax.fori_loop(0, 8, zchunk, 0)

    plsc.subcore_barrier()

    tile_base = s * TILE_EDGES
    bufs = ((src0, dstr0, dst0, vals0, rows0, isem0, gsem0),
            (src1, dstr1, dst1, vals1, rows1, isem1, gsem1))
    NCH = FULL_CH + 1

    def chunk_addr(j):
        return jnp.where(j < FULL_CH,
                         tile_base + j * CH,
                         jnp.where(s < REM_CH, NS * TILE_EDGES + s * CH, 0))

    def emit(j, par):
        srcA, dstrA, dstA, valsA, rowsA, isemA, gsemA = bufs[par]
        srcB, dstrB, dstB, valsB, rowsB, isemB, gsemB = bufs[1 - par]

        @pl.when((j >= 2) & (j <= NCH + 1))
        def _():
            pltpu.make_async_copy(emb_h.at[srcA], rowsA, gsemA).wait()

            def scale_group(g, _):
                vg = valsA[pl.ds(g * 16, 16)]
                for t in range(16):
                    v16 = lax.gather(
                        vg, jnp.full((16, 1), t, jnp.int32),
                        lax.GatherDimensionNumbers(offset_dims=(),
                                                   collapsed_slice_dims=(0,),
                                                   start_index_map=(0,)),
                        slice_sizes=(1,),
                        mode=lax.GatherScatterMode.PROMISE_IN_BOUNDS)
                    r = g * 16 + t
                    for q in range(4):
                        rowsA[r, pl.ds(q * 16, 16)] = (
                            rowsA[r, pl.ds(q * 16, 16)] * v16)
                return 0
            lax.fori_loop(0, CH // 16, scale_group, 0)
            pltpu.sync_copy(rowsA, acc.at[dstA], add=True)

        @pl.when(j <= NCH - 1)
        def _():
            cb = chunk_addr(j)
            pltpu.make_async_copy(src_h.at[pl.ds(cb, CH)], srcA, isemA).start()
            pltpu.make_async_copy(dst_h.at[pl.ds(cb, CH)], dstrA, isemA).start()
            pltpu.make_async_copy(vals_h.at[pl.ds(cb, CH)], valsA, isemA).start()

        @pl.when((j >= 1) & (j <= NCH))
        def _():
            cb = chunk_addr(j - 1)
            pltpu.make_async_copy(src_h.at[pl.ds(cb, CH)], srcB, isemB).wait()
            pltpu.make_async_copy(dst_h.at[pl.ds(cb, CH)], dstrB, isemB).wait()
            pltpu.make_async_copy(vals_h.at[pl.ds(cb, CH)], valsB, isemB).wait()
            dummy = (j - 1 == FULL_CH) & (s >= REM_CH)
            bound = jnp.where(dummy, 0, HALF_NODES)
            for k in range(CH // 16):
                d = dstrB[pl.ds(k * 16, 16)] - noff
                ok = (d >= 0) & (d < bound)
                dstB[pl.ds(k * 16, 16)] = jnp.where(ok, d, TRASH)
            pltpu.make_async_copy(emb_h.at[srcB], rowsB, gsemB).start()

    def pipe_body(jj, _):
        emit(2 * jj, 0)
        emit(2 * jj + 1, 1)
        return 0
    lax.fori_loop(0, 197, pipe_body, 0)
    plsc.subcore_barrier()

    @pl.when(s < 15)
    def _():
        pltpu.sync_copy(acc.at[pl.ds(s * 1560, 1560)],
                        out_h.at[pl.ds(noff + s * 1560, 1560)])

    @pl.when(s == 15)
    def _():
        pltpu.sync_copy(acc.at[pl.ds(15 * 1560, 1600)],
                        out_h.at[pl.ds(noff + 15 * 1560, 1600)])


_B_W = BATCH // (NC * NS)


@functools.partial(
    pl.kernel,
    out_type=[jax.ShapeDtypeStruct((BATCH, EMB), jnp.float32) for _ in range(6)],
    mesh=_mesh,
    scratch_types=[
        pltpu.VMEM((_B_W,), jnp.int32),
        pltpu.VMEM((_B_W,), jnp.int32),
        pltpu.VMEM((_B_W, EMB), jnp.float32),
        pltpu.VMEM((_B_W, EMB), jnp.float32),
        pltpu.VMEM((_B_W, EMB), jnp.float32),
        pltpu.VMEM((_B_W, EMB), jnp.float32),
        pltpu.SemaphoreType.DMA,
    ],
    compiler_params=pltpu.CompilerParams(use_tc_tiling_on_sc=False),
)
def _gather_mean(t0, t1, t2, t3, user_h, pos_h, neg_h,
                 ue_h, pe_h, ne_h, eu_h, ep_h, en_h,
                 idxr, idx, r0, r1, r2, r3, sem):
    c = lax.axis_index("c")
    s = lax.axis_index("s")
    wid = s * NC + c
    base = wid * _B_W

    for idx_h, off, mean_h, ego_h in ((user_h, 0, ue_h, eu_h),
                                      (pos_h, N_USERS, pe_h, ep_h),
                                      (neg_h, N_USERS, ne_h, en_h)):
        pltpu.sync_copy(idx_h.at[pl.ds(base, _B_W)], idxr)
        for k in range(_B_W // 16):
            idx[pl.ds(k * 16, 16)] = idxr[pl.ds(k * 16, 16)] + off
        pltpu.async_copy(t0.at[idx], r0, sem).wait()
        pltpu.async_copy(t1.at[idx], r1, sem).wait()
        pltpu.async_copy(t2.at[idx], r2, sem).wait()
        pltpu.async_copy(t3.at[idx], r3, sem).wait()
        pltpu.sync_copy(r0, ego_h.at[pl.ds(base, _B_W)])

        def mean_row(r, _):
            for q in range(4):
                sl = pl.ds(q * 16, 16)
                r0[r, sl] = (r0[r, sl] + r1[r, sl] + r2[r, sl] + r3[r, sl]) * 0.25
            return 0
        lax.fori_loop(0, _B_W, mean_row, 0)
        pltpu.sync_copy(r0, mean_h.at[pl.ds(base, _B_W)])


_RB = 256
_NBLK = BATCH // _RB


def _loss_body(ue_ref, pe_ref, ne_ref, eu_ref, ep_ref, en_ref,
               bpr_ref, reg_ref, na_ref):
    i = pl.program_id(0)
    ue_i = ue_ref[pl.ds(i * _RB, _RB), :]
    pe_i = pe_ref[pl.ds(i * _RB, _RB), :]
    ne_i = ne_ref[pl.ds(i * _RB, _RB), :]

    pos_s = jnp.sum(ue_i * pe_i, axis=-1)
    neg_s = jnp.sum(ue_i * ne_i, axis=-1)
    bpr_part = jnp.sum(jax.nn.softplus(neg_s - pos_s)) * (1.0 / BATCH)

    eu_i = eu_ref[pl.ds(i * _RB, _RB), :]
    ep_i = ep_ref[pl.ds(i * _RB, _RB), :]
    en_i = en_ref[pl.ds(i * _RB, _RB), :]
    reg_part = (jnp.sum(eu_i * eu_i) + jnp.sum(ep_i * ep_i) +
                jnp.sum(en_i * en_i)) * (REG_LAMBDA * 0.5 / BATCH)

    ue_all = ue_ref[...]
    pe_all = pe_ref[...]
    e1f = ue_all / jnp.maximum(
        jnp.sqrt(jnp.sum(ue_all * ue_all, axis=-1, keepdims=True)), 1e-12)
    e2f = pe_all / jnp.maximum(
        jnp.sqrt(jnp.sum(pe_all * pe_all, axis=-1, keepdims=True)), 1e-12)
    e1_i = ue_i / jnp.maximum(
        jnp.sqrt(jnp.sum(ue_i * ue_i, axis=-1, keepdims=True)), 1e-12)
    e2_i = pe_i / jnp.maximum(
        jnp.sqrt(jnp.sum(pe_i * pe_i, axis=-1, keepdims=True)), 1e-12)
    s1 = lax.dot_general(e1_i, e2f, (((1,), (1,)), ((), ())),
                         preferred_element_type=jnp.float32)
    s2 = lax.dot_general(e1_i, e1f, (((1,), (1,)), ((), ())),
                         preferred_element_type=jnp.float32)
    total = jnp.sum(jnp.exp((s1 + s2) * (1.0 / TAU)), axis=1)
    pos_sc = jnp.exp(jnp.sum(e1_i * e2_i, axis=-1) * (1.0 / TAU))
    na_part = jnp.sum(-jnp.log(pos_sc / total + 1e-5)) * (SSL_LAMBDA / BATCH)

    @pl.when(i == 0)
    def _():
        bpr_ref[0, 0] = 0.0
        reg_ref[0, 0] = 0.0
        na_ref[0, 0] = 0.0

    bpr_ref[0, 0] += bpr_part
    reg_ref[0, 0] += reg_part
    na_ref[0, 0] += na_part


def _losses(ue, pe, ne, eu, ep, en):
    full = pl.BlockSpec((BATCH, EMB), lambda i: (0, 0))
    scalar = pl.BlockSpec((1, 1), lambda i: (0, 0),
                          memory_space=pltpu.MemorySpace.SMEM)
    return pl.pallas_call(
        _loss_body,
        grid=(_NBLK,),
        in_specs=[full] * 6,
        out_specs=[scalar] * 3,
        out_shape=[jax.ShapeDtypeStruct((1, 1), jnp.float32)] * 3,
    )(ue, pe, ne, eu, ep, en)


def kernel(user, positive, negative, user_table, item_table, edge_index, edge_vals):
    t0 = jnp.concatenate([user_table, item_table], axis=0)
    src = edge_index[0]
    dst = edge_index[1]
    t1 = _propagate(t0, src, dst, edge_vals)
    t2 = _propagate(t1, src, dst, edge_vals)
    t3 = _propagate(t2, src, dst, edge_vals)
    ue, pe, ne, eu, ep, en = _gather_mean(t0, t1, t2, t3, user, positive, negative)
    bpr, reg, na = _losses(ue, pe, ne, eu, ep, en)
    return (bpr[0, 0], reg[0, 0], na[0, 0])


# speedup vs baseline: 1.1064x; 1.1064x over previous
import functools
import jax
import jax.numpy as jnp
from jax import lax
from jax.experimental import pallas as pl
from jax.experimental.pallas import tpu as pltpu
from jax.experimental.pallas import tpu_sc as plsc

N_USERS = 30000
N_ITEMS = 20000
N_NODES = N_USERS + N_ITEMS
EMB = 64
N_EDGE_HALF = 400000
BATCH = 4096
TAU = 0.2
REG_LAMBDA = 1e-4
SSL_LAMBDA = 0.1

NC, NS = 2, 16
CH = 128
N_EDGES = 2 * N_EDGE_HALF
FULL_CH = 390
TILE_EDGES = 49920
REM_CH = 10
HALF_NODES = 25000
ACC_ROWS = 25024
TRASH = 25000
ZR = 200

_mesh = plsc.VectorSubcoreMesh(core_axis_name="c", subcore_axis_name="s")


@functools.partial(
    pl.kernel,
    out_type=jax.ShapeDtypeStruct((N_NODES, EMB), jnp.float32),
    mesh=_mesh,
    scratch_types=[
        pltpu.VMEM((CH,), jnp.int32),
        pltpu.VMEM((CH,), jnp.int32),
        pltpu.VMEM((CH,), jnp.int32),
        pltpu.VMEM((CH,), jnp.float32),
        pltpu.VMEM((CH, EMB), jnp.float32),
        pltpu.VMEM((CH,), jnp.int32),
        pltpu.VMEM((CH,), jnp.int32),
        pltpu.VMEM((CH,), jnp.int32),
        pltpu.VMEM((CH,), jnp.float32),
        pltpu.VMEM((CH, EMB), jnp.float32),
        pltpu.VMEM((ZR, EMB), jnp.float32),
        pltpu.VMEM_SHARED((ACC_ROWS, EMB), jnp.float32),
        pltpu.SemaphoreType.DMA,
        pltpu.SemaphoreType.DMA,
        pltpu.SemaphoreType.DMA,
        pltpu.SemaphoreType.DMA,
        pltpu.SemaphoreType.DMA,
        pltpu.SemaphoreType.DMA,
    ],
    compiler_params=pltpu.CompilerParams(use_tc_tiling_on_sc=False),
)
def _propagate(emb_h, src_h, dst_h, vals_h, out_h,
               src0, dstr0, dst0, vals0, rows0,
               src1, dstr1, dst1, vals1, rows1,
               zbuf, acc, isem0, isem1, gsem0, gsem1, ssem0, ssem1):
    c = lax.axis_index("c")
    s = lax.axis_index("s")
    noff = c * HALF_NODES

    def zinit(r, _):
        for q in range(4):
            zbuf[r, pl.ds(q * 16, 16)] = jnp.zeros((16,), jnp.float32)
        return 0
    lax.fori_loop(0, ZR, zinit, 0)

    def zchunk(j, _):
        k = s + j * NS

        @pl.when(k < HALF_NODES // ZR)
        def _():
            pltpu.sync_copy(zbuf, acc.at[pl.ds(k * ZR, ZR)])
        return 0
    lax.fori_loop(0, 8, zchunk, 0)

    plsc.subcore_barrier()

    tile_base = s * TILE_EDGES
    bufs = ((src0, dstr0, dst0, vals0, rows0, isem0, gsem0, ssem0),
            (src1, dstr1, dst1, vals1, rows1, isem1, gsem1, ssem1))
    NCH = FULL_CH + 1

    def chunk_addr(j):
        return jnp.where(j < FULL_CH,
                         tile_base + j * CH,
                         jnp.where(s < REM_CH, NS * TILE_EDGES + s * CH, 0))

    def emit(j, par):
        srcA, dstrA, dstA, valsA, rowsA, isemA, gsemA, ssemA = bufs[par]
        srcB, dstrB, dstB, valsB, rowsB, isemB, gsemB, ssemB = bufs[1 - par]

        @pl.when((j >= 2) & (j <= NCH + 1))
        def _():
            pltpu.make_async_copy(emb_h.at[srcA], rowsA, gsemA).wait()

            def scale_group(g, _):
                vg = valsA[pl.ds(g * 16, 16)]
                for t in range(16):
                    v16 = lax.gather(
                        vg, jnp.full((16, 1), t, jnp.int32),
                        lax.GatherDimensionNumbers(offset_dims=(),
                                                   collapsed_slice_dims=(0,),
                                                   start_index_map=(0,)),
                        slice_sizes=(1,),
                        mode=lax.GatherScatterMode.PROMISE_IN_BOUNDS)
                    r = g * 16 + t
                    for q in range(4):
                        rowsA[r, pl.ds(q * 16, 16)] = (
                            rowsA[r, pl.ds(q * 16, 16)] * v16)
                return 0
            lax.fori_loop(0, CH // 16, scale_group, 0)
            pltpu.async_copy(rowsA, acc.at[dstA], ssemA, add=True)

        @pl.when(j <= NCH - 1)
        def _():
            cb = chunk_addr(j)
            pltpu.make_async_copy(src_h.at[pl.ds(cb, CH)], srcA, isemA).start()
            pltpu.make_async_copy(dst_h.at[pl.ds(cb, CH)], dstrA, isemA).start()
            pltpu.make_async_copy(vals_h.at[pl.ds(cb, CH)], valsA, isemA).start()

        @pl.when((j >= 1) & (j <= NCH))
        def _():
            cb = chunk_addr(j - 1)
            pltpu.make_async_copy(src_h.at[pl.ds(cb, CH)], srcB, isemB).wait()
            pltpu.make_async_copy(dst_h.at[pl.ds(cb, CH)], dstrB, isemB).wait()
            pltpu.make_async_copy(vals_h.at[pl.ds(cb, CH)], valsB, isemB).wait()
            dummy = (j - 1 == FULL_CH) & (s >= REM_CH)
            bound = jnp.where(dummy, 0, HALF_NODES)
            for k in range(CH // 16):
                d = dstrB[pl.ds(k * 16, 16)] - noff
                ok = (d >= 0) & (d < bound)
                dstB[pl.ds(k * 16, 16)] = jnp.where(ok, d, TRASH)
            @pl.when(j >= 3)
            def _():
                pltpu.make_async_copy(rowsB, acc.at[dstB], ssemB).wait()
            pltpu.make_async_copy(emb_h.at[srcB], rowsB, gsemB).start()

    def pipe_body(jj, _):
        emit(2 * jj, 0)
        emit(2 * jj + 1, 1)
        return 0
    lax.fori_loop(0, 197, pipe_body, 0)
    pltpu.make_async_copy(rows1, acc.at[dst1], ssem1).wait()
    pltpu.make_async_copy(rows0, acc.at[dst0], ssem0).wait()
    plsc.subcore_barrier()

    @pl.when(s < 15)
    def _():
        pltpu.sync_copy(acc.at[pl.ds(s * 1560, 1560)],
                        out_h.at[pl.ds(noff + s * 1560, 1560)])

    @pl.when(s == 15)
    def _():
        pltpu.sync_copy(acc.at[pl.ds(15 * 1560, 1600)],
                        out_h.at[pl.ds(noff + 15 * 1560, 1600)])


_B_W = BATCH // (NC * NS)


@functools.partial(
    pl.kernel,
    out_type=[jax.ShapeDtypeStruct((BATCH, EMB), jnp.float32) for _ in range(6)],
    mesh=_mesh,
    scratch_types=[
        pltpu.VMEM((_B_W,), jnp.int32),
        pltpu.VMEM((_B_W,), jnp.int32),
        pltpu.VMEM((_B_W, EMB), jnp.float32),
        pltpu.VMEM((_B_W, EMB), jnp.float32),
        pltpu.VMEM((_B_W, EMB), jnp.float32),
        pltpu.VMEM((_B_W, EMB), jnp.float32),
        pltpu.SemaphoreType.DMA,
    ],
    compiler_params=pltpu.CompilerParams(use_tc_tiling_on_sc=False),
)
def _gather_mean(t0, t1, t2, t3, user_h, pos_h, neg_h,
                 ue_h, pe_h, ne_h, eu_h, ep_h, en_h,
                 idxr, idx, r0, r1, r2, r3, sem):
    c = lax.axis_index("c")
    s = lax.axis_index("s")
    wid = s * NC + c
    base = wid * _B_W

    for idx_h, off, mean_h, ego_h in ((user_h, 0, ue_h, eu_h),
                                      (pos_h, N_USERS, pe_h, ep_h),
                                      (neg_h, N_USERS, ne_h, en_h)):
        pltpu.sync_copy(idx_h.at[pl.ds(base, _B_W)], idxr)
        for k in range(_B_W // 16):
            idx[pl.ds(k * 16, 16)] = idxr[pl.ds(k * 16, 16)] + off
        pltpu.async_copy(t0.at[idx], r0, sem).wait()
        pltpu.async_copy(t1.at[idx], r1, sem).wait()
        pltpu.async_copy(t2.at[idx], r2, sem).wait()
        pltpu.async_copy(t3.at[idx], r3, sem).wait()
        pltpu.sync_copy(r0, ego_h.at[pl.ds(base, _B_W)])

        def mean_row(r, _):
            for q in range(4):
                sl = pl.ds(q * 16, 16)
                r0[r, sl] = (r0[r, sl] + r1[r, sl] + r2[r, sl] + r3[r, sl]) * 0.25
            return 0
        lax.fori_loop(0, _B_W, mean_row, 0)
        pltpu.sync_copy(r0, mean_h.at[pl.ds(base, _B_W)])


_RB = 256
_NBLK = BATCH // _RB


def _loss_body(ue_ref, pe_ref, ne_ref, eu_ref, ep_ref, en_ref,
               bpr_ref, reg_ref, na_ref):
    i = pl.program_id(0)
    ue_i = ue_ref[pl.ds(i * _RB, _RB), :]
    pe_i = pe_ref[pl.ds(i * _RB, _RB), :]
    ne_i = ne_ref[pl.ds(i * _RB, _RB), :]

    pos_s = jnp.sum(ue_i * pe_i, axis=-1)
    neg_s = jnp.sum(ue_i * ne_i, axis=-1)
    bpr_part = jnp.sum(jax.nn.softplus(neg_s - pos_s)) * (1.0 / BATCH)

    eu_i = eu_ref[pl.ds(i * _RB, _RB), :]
    ep_i = ep_ref[pl.ds(i * _RB, _RB), :]
    en_i = en_ref[pl.ds(i * _RB, _RB), :]
    reg_part = (jnp.sum(eu_i * eu_i) + jnp.sum(ep_i * ep_i) +
                jnp.sum(en_i * en_i)) * (REG_LAMBDA * 0.5 / BATCH)

    ue_all = ue_ref[...]
    pe_all = pe_ref[...]
    e1f = ue_all / jnp.maximum(
        jnp.sqrt(jnp.sum(ue_all * ue_all, axis=-1, keepdims=True)), 1e-12)
    e2f = pe_all / jnp.maximum(
        jnp.sqrt(jnp.sum(pe_all * pe_all, axis=-1, keepdims=True)), 1e-12)
    e1_i = ue_i / jnp.maximum(
        jnp.sqrt(jnp.sum(ue_i * ue_i, axis=-1, keepdims=True)), 1e-12)
    e2_i = pe_i / jnp.maximum(
        jnp.sqrt(jnp.sum(pe_i * pe_i, axis=-1, keepdims=True)), 1e-12)
    s1 = lax.dot_general(e1_i, e2f, (((1,), (1,)), ((), ())),
                         preferred_element_type=jnp.float32)
    s2 = lax.dot_general(e1_i, e1f, (((1,), (1,)), ((), ())),
                         preferred_element_type=jnp.float32)
    total = jnp.sum(jnp.exp((s1 + s2) * (1.0 / TAU)), axis=1)
    pos_sc = jnp.exp(jnp.sum(e1_i * e2_i, axis=-1) * (1.0 / TAU))
    na_part = jnp.sum(-jnp.log(pos_sc / total + 1e-5)) * (SSL_LAMBDA / BATCH)

    @pl.when(i == 0)
    def _():
        bpr_ref[0, 0] = 0.0
        reg_ref[0, 0] = 0.0
        na_ref[0, 0] = 0.0

    bpr_ref[0, 0] += bpr_part
    reg_ref[0, 0] += reg_part
    na_ref[0, 0] += na_part


def _losses(ue, pe, ne, eu, ep, en):
    full = pl.BlockSpec((BATCH, EMB), lambda i: (0, 0))
    scalar = pl.BlockSpec((1, 1), lambda i: (0, 0),
                          memory_space=pltpu.MemorySpace.SMEM)
    return pl.pallas_call(
        _loss_body,
        grid=(_NBLK,),
        in_specs=[full] * 6,
        out_specs=[scalar] * 3,
        out_shape=[jax.ShapeDtypeStruct((1, 1), jnp.float32)] * 3,
    )(ue, pe, ne, eu, ep, en)


def kernel(user, positive, negative, user_table, item_table, edge_index, edge_vals):
    t0 = jnp.concatenate([user_table, item_table], axis=0)
    src = edge_index[0]
    dst = edge_index[1]
    t1 = _propagate(t0, src, dst, edge_vals)
    t2 = _propagate(t1, src, dst, edge_vals)
    t3 = _propagate(t2, src, dst, edge_vals)
    ue, pe, ne, eu, ep, en = _gather_mean(t0, t1, t2, t3, user, positive, negative)
    bpr, reg, na = _losses(ue, pe, ne, eu, ep, en)
    return (bpr[0, 0], reg[0, 0], na[0, 0])


# speedup vs baseline: 2.1171x; 1.9135x over previous
import functools
import jax
import jax.numpy as jnp
from jax import lax
from jax.experimental import pallas as pl
from jax.experimental.pallas import tpu as pltpu
from jax.experimental.pallas import tpu_sc as plsc

N_USERS = 30000
N_ITEMS = 20000
N_NODES = N_USERS + N_ITEMS
EMB = 64
N_EDGE_HALF = 400000
BATCH = 4096
TAU = 0.2
REG_LAMBDA = 1e-4
SSL_LAMBDA = 0.1

NC, NS = 2, 16
CH = 128
N_EDGES = 2 * N_EDGE_HALF
FULL_CH = 390
TILE_EDGES = 49920
REM_CH = 10
HALF_NODES = 25000
ACC_ROWS = 25024
TRASH = 25000
ZR = 200

_mesh = plsc.VectorSubcoreMesh(core_axis_name="c", subcore_axis_name="s")


@functools.partial(
    pl.kernel,
    out_type=jax.ShapeDtypeStruct((N_NODES, EMB), jnp.float32),
    mesh=_mesh,
    scratch_types=[
        pltpu.VMEM((CH,), jnp.int32),
        pltpu.VMEM((CH,), jnp.int32),
        pltpu.VMEM((CH,), jnp.int32),
        pltpu.VMEM((CH,), jnp.float32),
        pltpu.VMEM((CH, EMB), jnp.float32),
        pltpu.VMEM((CH,), jnp.int32),
        pltpu.VMEM((CH,), jnp.int32),
        pltpu.VMEM((CH,), jnp.int32),
        pltpu.VMEM((CH,), jnp.float32),
        pltpu.VMEM((CH, EMB), jnp.float32),
        pltpu.VMEM((ZR, EMB), jnp.float32),
        pltpu.VMEM_SHARED((ACC_ROWS, EMB), jnp.float32),
        pltpu.SemaphoreType.DMA,
        pltpu.SemaphoreType.DMA,
        pltpu.SemaphoreType.DMA,
        pltpu.SemaphoreType.DMA,
        pltpu.SemaphoreType.DMA,
        pltpu.SemaphoreType.DMA,
    ],
    compiler_params=pltpu.CompilerParams(use_tc_tiling_on_sc=False),
)
def _propagate(emb_h, src_h, dst_h, vals_h, out_h,
               src0, dstr0, dst0, vals0, rows0,
               src1, dstr1, dst1, vals1, rows1,
               zbuf, acc, isem0, isem1, gsem0, gsem1, ssem0, ssem1):
    c = lax.axis_index("c")
    s = lax.axis_index("s")
    noff = c * HALF_NODES

    def zinit(r, _):
        for q in range(4):
            zbuf[r, pl.ds(q * 16, 16)] = jnp.zeros((16,), jnp.float32)
        return 0
    lax.fori_loop(0, ZR, zinit, 0)

    def zchunk(j, _):
        k = s + j * NS

        @pl.when(k < HALF_NODES // ZR)
        def _():
            pltpu.sync_copy(zbuf, acc.at[pl.ds(k * ZR, ZR)])
        return 0
    lax.fori_loop(0, 8, zchunk, 0)

    plsc.subcore_barrier()

    tile_base = s * TILE_EDGES
    bufs = ((src0, dstr0, dst0, vals0, rows0, isem0, gsem0, ssem0),
            (src1, dstr1, dst1, vals1, rows1, isem1, gsem1, ssem1))
    NCH = FULL_CH + 1

    def chunk_addr(j):
        return jnp.where(j < FULL_CH,
                         tile_base + j * CH,
                         jnp.where(s < REM_CH, NS * TILE_EDGES + s * CH, 0))

    def emit(j, par):
        srcA, dstrA, dstA, valsA, rowsA, isemA, gsemA, ssemA = bufs[par]
        srcB, dstrB, dstB, valsB, rowsB, isemB, gsemB, ssemB = bufs[1 - par]

        @pl.when((j >= 2) & (j <= NCH + 1))
        def _():
            pltpu.make_async_copy(emb_h.at[srcA], rowsA, gsemA).wait()

            def scale_group(g, _):
                vg = valsA[pl.ds(g * 16, 16)]
                for t in range(16):
                    v16 = lax.gather(
                        vg, jnp.full((16, 1), t, jnp.int32),
                        lax.GatherDimensionNumbers(offset_dims=(),
                                                   collapsed_slice_dims=(0,),
                                                   start_index_map=(0,)),
                        slice_sizes=(1,),
                        mode=lax.GatherScatterMode.PROMISE_IN_BOUNDS)
                    r = g * 16 + t
                    for q in range(4):
                        rowsA[r, pl.ds(q * 16, 16)] = (
                            rowsA[r, pl.ds(q * 16, 16)] * v16)
                return 0
            def scale_pair(g2, _):
                scale_group(2 * g2, 0)
                scale_group(2 * g2 + 1, 0)
                return 0
            lax.fori_loop(0, CH // 32, scale_pair, 0)
            pltpu.async_copy(rowsA, acc.at[dstA], ssemA, add=True)

        @pl.when(j <= NCH - 1)
        def _():
            cb = chunk_addr(j)
            pltpu.make_async_copy(src_h.at[pl.ds(cb, CH)], srcA, isemA).start()
            pltpu.make_async_copy(dst_h.at[pl.ds(cb, CH)], dstrA, isemA).start()
            pltpu.make_async_copy(vals_h.at[pl.ds(cb, CH)], valsA, isemA).start()

        @pl.when((j >= 1) & (j <= NCH))
        def _():
            cb = chunk_addr(j - 1)
            pltpu.make_async_copy(src_h.at[pl.ds(cb, CH)], srcB, isemB).wait()
            pltpu.make_async_copy(dst_h.at[pl.ds(cb, CH)], dstrB, isemB).wait()
            pltpu.make_async_copy(vals_h.at[pl.ds(cb, CH)], valsB, isemB).wait()
            dummy = (j - 1 == FULL_CH) & (s >= REM_CH)
            bound = jnp.where(dummy, 0, HALF_NODES)
            for k in range(CH // 16):
                d = dstrB[pl.ds(k * 16, 16)] - noff
                ok = (d >= 0) & (d < bound)
                dstB[pl.ds(k * 16, 16)] = jnp.where(ok, d, TRASH)
            @pl.when(j >= 3)
            def _():
                pltpu.make_async_copy(rowsB, acc.at[dstB], ssemB).wait()
            pltpu.make_async_copy(emb_h.at[srcB], rowsB, gsemB).start()

    def pipe_body(jj, _):
        emit(2 * jj, 0)
        emit(2 * jj + 1, 1)
        return 0
    lax.fori_loop(0, 197, pipe_body, 0)
    pltpu.make_async_copy(rows1, acc.at[dst1], ssem1).wait()
    pltpu.make_async_copy(rows0, acc.at[dst0], ssem0).wait()
    plsc.subcore_barrier()

    @pl.when(s < 15)
    def _():
        pltpu.sync_copy(acc.at[pl.ds(s * 1560, 1560)],
                        out_h.at[pl.ds(noff + s * 1560, 1560)])

    @pl.when(s == 15)
    def _():
        pltpu.sync_copy(acc.at[pl.ds(15 * 1560, 1600)],
                        out_h.at[pl.ds(noff + 15 * 1560, 1600)])


_B_W = BATCH // (NC * NS)


@functools.partial(
    pl.kernel,
    out_type=[jax.ShapeDtypeStruct((BATCH, EMB), jnp.float32) for _ in range(6)],
    mesh=_mesh,
    scratch_types=[
        pltpu.VMEM((_B_W,), jnp.int32),
        pltpu.VMEM((_B_W,), jnp.int32),
        pltpu.VMEM((_B_W, EMB), jnp.float32),
        pltpu.VMEM((_B_W, EMB), jnp.float32),
        pltpu.VMEM((_B_W, EMB), jnp.float32),
        pltpu.VMEM((_B_W, EMB), jnp.float32),
        pltpu.SemaphoreType.DMA,
    ],
    compiler_params=pltpu.CompilerParams(use_tc_tiling_on_sc=False),
)
def _gather_mean(t0, t1, t2, t3, user_h, pos_h, neg_h,
                 ue_h, pe_h, ne_h, eu_h, ep_h, en_h,
                 idxr, idx, r0, r1, r2, r3, sem):
    c = lax.axis_index("c")
    s = lax.axis_index("s")
    wid = s * NC + c
    base = wid * _B_W

    for idx_h, off, mean_h, ego_h in ((user_h, 0, ue_h, eu_h),
                                      (pos_h, N_USERS, pe_h, ep_h),
                                      (neg_h, N_USERS, ne_h, en_h)):
        pltpu.sync_copy(idx_h.at[pl.ds(base, _B_W)], idxr)
        for k in range(_B_W // 16):
            idx[pl.ds(k * 16, 16)] = idxr[pl.ds(k * 16, 16)] + off
        pltpu.async_copy(t0.at[idx], r0, sem).wait()
        pltpu.async_copy(t1.at[idx], r1, sem).wait()
        pltpu.async_copy(t2.at[idx], r2, sem).wait()
        pltpu.async_copy(t3.at[idx], r3, sem).wait()
        pltpu.sync_copy(r0, ego_h.at[pl.ds(base, _B_W)])

        def mean_row(r, _):
            for q in range(4):
                sl = pl.ds(q * 16, 16)
                r0[r, sl] = (r0[r, sl] + r1[r, sl] + r2[r, sl] + r3[r, sl]) * 0.25
            return 0
        lax.fori_loop(0, _B_W, mean_row, 0)
        pltpu.sync_copy(r0, mean_h.at[pl.ds(base, _B_W)])


_RB = 256
_NBLK = BATCH // _RB


def _loss_body(ue_ref, pe_ref, ne_ref, eu_ref, ep_ref, en_ref,
               bpr_ref, reg_ref, na_ref):
    i = pl.program_id(0)
    ue_i = ue_ref[pl.ds(i * _RB, _RB), :]
    pe_i = pe_ref[pl.ds(i * _RB, _RB), :]
    ne_i = ne_ref[pl.ds(i * _RB, _RB), :]

    pos_s = jnp.sum(ue_i * pe_i, axis=-1)
    neg_s = jnp.sum(ue_i * ne_i, axis=-1)
    bpr_part = jnp.sum(jax.nn.softplus(neg_s - pos_s)) * (1.0 / BATCH)

    eu_i = eu_ref[pl.ds(i * _RB, _RB), :]
    ep_i = ep_ref[pl.ds(i * _RB, _RB), :]
    en_i = en_ref[pl.ds(i * _RB, _RB), :]
    reg_part = (jnp.sum(eu_i * eu_i) + jnp.sum(ep_i * ep_i) +
                jnp.sum(en_i * en_i)) * (REG_LAMBDA * 0.5 / BATCH)

    ue_all = ue_ref[...]
    pe_all = pe_ref[...]
    e1f = ue_all / jnp.maximum(
        jnp.sqrt(jnp.sum(ue_all * ue_all, axis=-1, keepdims=True)), 1e-12)
    e2f = pe_all / jnp.maximum(
        jnp.sqrt(jnp.sum(pe_all * pe_all, axis=-1, keepdims=True)), 1e-12)
    e1_i = ue_i / jnp.maximum(
        jnp.sqrt(jnp.sum(ue_i * ue_i, axis=-1, keepdims=True)), 1e-12)
    e2_i = pe_i / jnp.maximum(
        jnp.sqrt(jnp.sum(pe_i * pe_i, axis=-1, keepdims=True)), 1e-12)
    s1 = lax.dot_general(e1_i, e2f, (((1,), (1,)), ((), ())),
                         preferred_element_type=jnp.float32)
    s2 = lax.dot_general(e1_i, e1f, (((1,), (1,)), ((), ())),
                         preferred_element_type=jnp.float32)
    total = jnp.sum(jnp.exp((s1 + s2) * (1.0 / TAU)), axis=1)
    pos_sc = jnp.exp(jnp.sum(e1_i * e2_i, axis=-1) * (1.0 / TAU))
    na_part = jnp.sum(-jnp.log(pos_sc / total + 1e-5)) * (SSL_LAMBDA / BATCH)

    @pl.when(i == 0)
    def _():
        bpr_ref[0, 0] = 0.0
        reg_ref[0, 0] = 0.0
        na_ref[0, 0] = 0.0

    bpr_ref[0, 0] += bpr_part
    reg_ref[0, 0] += reg_part
    na_ref[0, 0] += na_part


def _losses(ue, pe, ne, eu, ep, en):
    full = pl.BlockSpec((BATCH, EMB), lambda i: (0, 0))
    scalar = pl.BlockSpec((1, 1), lambda i: (0, 0),
                          memory_space=pltpu.MemorySpace.SMEM)
    return pl.pallas_call(
        _loss_body,
        grid=(_NBLK,),
        in_specs=[full] * 6,
        out_specs=[scalar] * 3,
        out_shape=[jax.ShapeDtypeStruct((1, 1), jnp.float32)] * 3,
    )(ue, pe, ne, eu, ep, en)


def kernel(user, positive, negative, user_table, item_table, edge_index, edge_vals):
    t0 = jnp.concatenate([user_table, item_table], axis=0)
    src = edge_index[0]
    dst = edge_index[1]
    t1 = _propagate(t0, src, dst, edge_vals)
    t2 = _propagate(t1, src, dst, edge_vals)
    t3 = _propagate(t2, src, dst, edge_vals)
    ue, pe, ne, eu, ep, en = _gather_mean(t0, t1, t2, t3, user, positive, negative)
    bpr, reg, na = _losses(ue, pe, ne, eu, ep, en)
    return (bpr[0, 0], reg[0, 0], na[0, 0])
